# Initial kernel scaffold; baseline (speedup 1.0000x reference)
import jax
import jax.numpy as jnp
from jax.experimental import pallas as pl


def kernel(in_feat, edge_index, W1, b1, W2, b2):
    raise NotImplementedError("write your pallas kernel here")


# speedup vs baseline: 12.0200x; 12.0200x over previous
import functools

import jax
import jax.numpy as jnp
from jax import lax
from jax.experimental import pallas as pl
from jax.experimental.pallas import tpu as pltpu
from jax.experimental.pallas import tpu_sc as plsc

N_NODES = 10000
N_EDGES = 320000
IN_FEATS = 128
H_FEATS = 16

NC = 2
NS = 16
NW = NC * NS
CHUNK = 128
NCHUNK = -(-N_EDGES // (NW * CHUNK))
EPT = NCHUNK * CHUNK
E_PAD = NW * EPT
N_PAD_SLOTS = 112
N_TOT = N_NODES + N_PAD_SLOTS
ROWS_PT = N_TOT // NS
HH = H_FEATS // 2
FLAT = H_FEATS * N_TOT
FLAT_PT = FLAT // NS

_MESH = plsc.VectorSubcoreMesh(core_axis_name="c", subcore_axis_name="s")


@functools.partial(
    pl.kernel,
    out_type=[
        jax.ShapeDtypeStruct((NC * N_TOT,), jnp.float32),
        jax.ShapeDtypeStruct((NC * N_TOT,), jnp.float32),
    ],
    mesh=_MESH,
    scratch_types=[
        pltpu.VMEM((NCHUNK, CHUNK), jnp.int32),
        pltpu.VMEM((NCHUNK, CHUNK), jnp.int32),
        pltpu.VMEM((CHUNK,), jnp.float32),
        pltpu.VMEM_SHARED((N_TOT,), jnp.float32),
        pltpu.VMEM_SHARED((N_TOT,), jnp.float32),
    ],
)
def _sc_degrees(src_hbm, dst_hbm, z1_hbm, ones_hbm, dout_hbm, din_hbm,
                src_v, dst_v, ones_v, dout_s, din_s):
    cid = lax.axis_index("c")
    sid = lax.axis_index("s")
    wid = sid * NC + cid

    pltpu.sync_copy(src_hbm.at[wid], src_v)
    pltpu.sync_copy(dst_hbm.at[wid], dst_v)
    pltpu.sync_copy(ones_hbm, ones_v)

    @pl.when(sid == 0)
    def _():
        pltpu.sync_copy(z1_hbm, dout_s)

    @pl.when(sid == 1)
    def _():
        pltpu.sync_copy(z1_hbm, din_s)

    plsc.subcore_barrier()

    def body(j, carry):
        pltpu.sync_copy(ones_v, dout_s.at[src_v.at[j]], add=True)
        pltpu.sync_copy(ones_v, din_s.at[dst_v.at[j]], add=True)
        return carry

    lax.fori_loop(0, NCHUNK, body, 0)
    plsc.subcore_barrier()

    @pl.when(sid == 0)
    def _():
        pltpu.sync_copy(dout_s, dout_hbm.at[pl.ds(cid * N_TOT, N_TOT)])

    @pl.when(sid == 1)
    def _():
        pltpu.sync_copy(din_s, din_hbm.at[pl.ds(cid * N_TOT, N_TOT)])


@functools.partial(
    pl.kernel,
    out_type=[
        jax.ShapeDtypeStruct((NC * FLAT,), jnp.float32),
        jax.ShapeDtypeStruct((NC * N_TOT,), jnp.float32),
    ],
    mesh=_MESH,
    scratch_types=[
        pltpu.VMEM((HH, CHUNK), jnp.int32),
        pltpu.VMEM((HH, CHUNK), jnp.int32),
        pltpu.VMEM((HH, CHUNK), jnp.float32),
        pltpu.VMEM((CHUNK,), jnp.float32),
        pltpu.VMEM_SHARED((FLAT,), jnp.float32),
        pltpu.VMEM_SHARED((N_TOT,), jnp.float32),
        pltpu.VMEM_SHARED((FLAT,), jnp.float32),
        pltpu.VMEM_SHARED((N_TOT,), jnp.float32),
        pltpu.SemaphoreType.DMA,
        pltpu.SemaphoreType.DMA,
    ],
)
def _sc_edge_pass(srck_hbm, dstk_hbm, h1st_hbm, nd_hbm, zf_hbm,
                  aggt_hbm, t_hbm,
                  idxs_v, idxd_v, cols_v, nval_v, h1st_s, nd_s, aggt_s, t_s,
                  sem_g, sem_s):
    cid = lax.axis_index("c")
    sid = lax.axis_index("s")
    wid = sid * NC + cid

    pltpu.sync_copy(zf_hbm.at[pl.ds(sid * FLAT_PT, FLAT_PT)],
                    aggt_s.at[pl.ds(sid * FLAT_PT, FLAT_PT)])
    pltpu.sync_copy(h1st_hbm.at[pl.ds(sid * FLAT_PT, FLAT_PT)],
                    h1st_s.at[pl.ds(sid * FLAT_PT, FLAT_PT)])

    @pl.when(sid == 0)
    def _():
        pltpu.sync_copy(zf_hbm.at[pl.ds(0, N_TOT)], t_s)

    @pl.when(sid == 1)
    def _():
        pltpu.sync_copy(nd_hbm, nd_s)

    plsc.subcore_barrier()

    def make_body(half):
        def body(j, carry):
            pltpu.sync_copy(srck_hbm.at[wid, j, pl.ds(half * HH, HH)], idxs_v)
            pltpu.sync_copy(dstk_hbm.at[wid, j, pl.ds(half * HH, HH)], idxd_v)
            gath = [pltpu.async_copy(h1st_s.at[idxs_v.at[k]], cols_v.at[k],
                                     sem_g) for k in range(HH)]
            if half == 0:
                gnd = pltpu.async_copy(nd_s.at[idxd_v.at[0]], nval_v, sem_g)
            for g in gath:
                g.wait()
            if half == 0:
                gnd.wait()
            scat = [pltpu.async_copy(cols_v.at[k], aggt_s.at[idxd_v.at[k]],
                                     sem_s, add=True) for k in range(HH)]
            if half == 0:
                st = pltpu.async_copy(nval_v, t_s.at[idxs_v.at[0]], sem_s,
                                      add=True)
            for s in scat:
                s.wait()
            if half == 0:
                st.wait()
            return carry
        return body

    lax.fori_loop(0, NCHUNK, make_body(0), 0)
    lax.fori_loop(0, NCHUNK, make_body(1), 0)
    plsc.subcore_barrier()

    pltpu.sync_copy(aggt_s.at[pl.ds(sid * FLAT_PT, FLAT_PT)],
                    aggt_hbm.at[pl.ds(cid * FLAT + sid * FLAT_PT, FLAT_PT)])

    @pl.when(sid == 0)
    def _():
        pltpu.sync_copy(t_s, t_hbm.at[pl.ds(cid * N_TOT, N_TOT)])


def _tc_matmul_body(x_ref, w_ref, o_ref):
    o_ref[...] = jnp.dot(x_ref[...], w_ref[...],
                         preferred_element_type=jnp.float32)


def _tc_norms_body(p_ref, do_ref, di_ref, h_ref, nd_ref, ns_ref):
    do = do_ref[0] + do_ref[1]
    di = di_ref[0] + di_ref[1]
    ns = jnp.where(do > 0.0, lax.rsqrt(do), 0.0)
    nd = jnp.where(di > 0.0, lax.rsqrt(di), 0.0)
    h_ref[...] = p_ref[...] * ns[:, None]
    nd_ref[...] = nd
    ns_ref[...] = ns


def _tc_final_body(aggt_ref, t_ref, nd_ref, ns_ref, b1_ref, w2_ref, b2_ref,
                   o_ref):
    aggt = aggt_ref[0] + aggt_ref[1]
    r = jnp.maximum(aggt * nd_ref[...][None, :] + b1_ref[...], 0.0)
    c = jnp.sum(r * w2_ref[...], axis=0) * ns_ref[...]
    t = t_ref[0] + t_ref[1]
    rowid = lax.broadcasted_iota(jnp.int32, (N_TOT,), 0)
    s = jnp.sum(jnp.where(rowid < N_NODES, c * t, 0.0))
    o_ref[...] = s.reshape(1, 1) / N_NODES + b2_ref[...]


def kernel(in_feat, edge_index, W1, b1, W2, b2):
    xp = jnp.pad(in_feat, ((0, N_TOT - N_NODES), (0, 0)))
    pad_idx = N_NODES + (jnp.arange(E_PAD - N_EDGES, dtype=jnp.int32)
                         % N_PAD_SLOTS)
    src_p = jnp.concatenate([edge_index[0], pad_idx])
    dst_p = jnp.concatenate([edge_index[1], pad_idx])
    src_r = src_p.reshape(NW, NCHUNK, CHUNK)
    dst_r = dst_p.reshape(NW, NCHUNK, CHUNK)
    shift = (jnp.arange(H_FEATS, dtype=jnp.int32) * N_TOT)[None, None, :, None]
    srck = (src_r[:, :, None, :] + shift).reshape(NW, NCHUNK, H_FEATS, CHUNK)
    dstk = (dst_r[:, :, None, :] + shift).reshape(NW, NCHUNK, H_FEATS, CHUNK)
    z1 = jnp.zeros((N_TOT,), jnp.float32)
    zf = jnp.zeros((FLAT,), jnp.float32)
    ones = jnp.ones((CHUNK,), jnp.float32)

    p = pl.pallas_call(
        _tc_matmul_body,
        out_shape=jax.ShapeDtypeStruct((N_TOT, H_FEATS), jnp.float32),
    )(xp, W1)

    dout_p, din_p = _sc_degrees(src_r, dst_r, z1, ones)

    h1s, nd, ns = pl.pallas_call(
        _tc_norms_body,
        out_shape=[
            jax.ShapeDtypeStruct((N_TOT, H_FEATS), jnp.float32),
            jax.ShapeDtypeStruct((N_TOT,), jnp.float32),
            jax.ShapeDtypeStruct((N_TOT,), jnp.float32),
        ],
    )(p, dout_p.reshape(NC, N_TOT), din_p.reshape(NC, N_TOT))

    h1st = h1s.T.reshape(FLAT)

    aggt_p, t_p = _sc_edge_pass(srck, dstk, h1st, nd, zf)

    out = pl.pallas_call(
        _tc_final_body,
        out_shape=jax.ShapeDtypeStruct((1, 1), jnp.float32),
    )(aggt_p.reshape(NC, H_FEATS, N_TOT), t_p.reshape(NC, N_TOT), nd, ns,
      b1.reshape(H_FEATS, 1), W2.reshape(H_FEATS, 1), b2.reshape(1, 1))
    return out


# speedup vs baseline: 15.0019x; 1.2481x over previous
import functools

import jax
import jax.numpy as jnp
from jax import lax
from jax.experimental import pallas as pl
from jax.experimental.pallas import tpu as pltpu
from jax.experimental.pallas import tpu_sc as plsc

N_NODES = 10000
N_EDGES = 320000
IN_FEATS = 128
H_FEATS = 16
NSEG = H_FEATS + 1

NC = 2
NS = 16
NW = NC * NS
CHUNK = 128
NCHUNK = 80
EPT = NCHUNK * CHUNK
E_PAD = NW * EPT
N_PAD_SLOTS = 112
N_TOT = N_NODES + N_PAD_SLOTS
FLAT = NSEG * N_TOT
FLAT_PAD = 172032
FLAT_PT = FLAT_PAD // NS
GA, GB = 8, 9

_MESH = plsc.VectorSubcoreMesh(core_axis_name="c", subcore_axis_name="s")


@functools.partial(
    pl.kernel,
    out_type=[
        jax.ShapeDtypeStruct((NC * N_TOT,), jnp.float32),
        jax.ShapeDtypeStruct((NC * N_TOT,), jnp.float32),
    ],
    mesh=_MESH,
    scratch_types=[
        pltpu.VMEM((NCHUNK, CHUNK), jnp.int32),
        pltpu.VMEM((NCHUNK, CHUNK), jnp.int32),
        pltpu.VMEM((CHUNK,), jnp.float32),
        pltpu.VMEM_SHARED((N_TOT,), jnp.float32),
        pltpu.VMEM_SHARED((N_TOT,), jnp.float32),
    ],
)
def _sc_degrees(src_hbm, dst_hbm, z1_hbm, ones_hbm, dout_hbm, din_hbm,
                src_v, dst_v, ones_v, dout_s, din_s):
    cid = lax.axis_index("c")
    sid = lax.axis_index("s")
    wid = sid * NC + cid

    pltpu.sync_copy(src_hbm.at[wid], src_v)
    pltpu.sync_copy(dst_hbm.at[wid], dst_v)
    pltpu.sync_copy(ones_hbm, ones_v)

    @pl.when(sid == 0)
    def _():
        pltpu.sync_copy(z1_hbm, dout_s)

    @pl.when(sid == 1)
    def _():
        pltpu.sync_copy(z1_hbm, din_s)

    plsc.subcore_barrier()

    def body(j, carry):
        pltpu.sync_copy(ones_v, dout_s.at[src_v.at[j]], add=True)
        pltpu.sync_copy(ones_v, din_s.at[dst_v.at[j]], add=True)
        return carry

    lax.fori_loop(0, NCHUNK, body, 0)
    plsc.subcore_barrier()

    @pl.when(sid == 0)
    def _():
        pltpu.sync_copy(dout_s, dout_hbm.at[pl.ds(cid * N_TOT, N_TOT)])

    @pl.when(sid == 1)
    def _():
        pltpu.sync_copy(din_s, din_hbm.at[pl.ds(cid * N_TOT, N_TOT)])


@functools.partial(
    pl.kernel,
    out_type=[
        jax.ShapeDtypeStruct((NC * FLAT_PAD,), jnp.float32),
    ],
    mesh=_MESH,
    scratch_types=[
        pltpu.VMEM((GB, CHUNK), jnp.int32),
        pltpu.VMEM((NCHUNK, 8, CHUNK), jnp.int32),
        pltpu.VMEM((NCHUNK, 1, CHUNK), jnp.int32),
        pltpu.VMEM((GB, CHUNK), jnp.float32),
        pltpu.VMEM_SHARED((FLAT_PAD,), jnp.float32),
        pltpu.VMEM_SHARED((FLAT_PAD,), jnp.float32),
        pltpu.SemaphoreType.DMA,
        pltpu.SemaphoreType.DMA,
        pltpu.SemaphoreType.DMA,
    ],
)
def _sc_edge_pass(gidx_hbm, sidx_hbm, tab_hbm, zf_hbm, agg_hbm,
                  idxs_v, bigd_v, bigdt_v, cols_v, tab_s, agg_s,
                  sem_i, sem_g, sem_s):
    cid = lax.axis_index("c")
    sid = lax.axis_index("s")
    wid = sid * NC + cid

    pltpu.sync_copy(zf_hbm.at[pl.ds(sid * FLAT_PT, FLAT_PT)],
                    agg_s.at[pl.ds(sid * FLAT_PT, FLAT_PT)])
    pltpu.sync_copy(tab_hbm.at[pl.ds(sid * FLAT_PT, FLAT_PT)],
                    tab_s.at[pl.ds(sid * FLAT_PT, FLAT_PT)])
    plsc.subcore_barrier()

    def run_group(off, size):
        pltpu.sync_copy(sidx_hbm.at[wid, :, pl.ds(off, 8)], bigd_v)
        if size == GB:
            pltpu.sync_copy(sidx_hbm.at[wid, :, pl.ds(NSEG - 1, 1)], bigdt_v)

        def issue_gidx(j):
            pltpu.async_copy(gidx_hbm.at[wid, j, pl.ds(off, 8)],
                             idxs_v.at[pl.ds(0, 8)], sem_i)
            if size == GB:
                pltpu.async_copy(gidx_hbm.at[wid, j, pl.ds(NSEG - 1, 1)],
                                 idxs_v.at[pl.ds(8, 1)], sem_i)

        def drain_gidx(j):
            pltpu.make_async_copy(gidx_hbm.at[wid, j, pl.ds(off, 8)],
                                  idxs_v.at[pl.ds(0, 8)], sem_i).wait()
            if size == GB:
                pltpu.make_async_copy(gidx_hbm.at[wid, j, pl.ds(NSEG - 1, 1)],
                                      idxs_v.at[pl.ds(8, 1)], sem_i).wait()

        issue_gidx(0)

        def body(j, carry):
            drain_gidx(j)
            gath = [pltpu.async_copy(tab_s.at[idxs_v.at[k]], cols_v.at[k],
                                     sem_g) for k in range(size)]
            for g in gath:
                g.wait()

            @pl.when(j < NCHUNK - 1)
            def _():
                issue_gidx(j + 1)

            scat = [pltpu.async_copy(
                cols_v.at[k],
                agg_s.at[bigd_v.at[j, k] if k < 8 else bigdt_v.at[j, 0]],
                sem_s, add=True) for k in range(size)]
            for s in scat:
                s.wait()
            return carry

        lax.fori_loop(0, NCHUNK, body, 0)

    run_group(0, GA)
    run_group(8, GB)
    plsc.subcore_barrier()

    pltpu.sync_copy(agg_s.at[pl.ds(sid * FLAT_PT, FLAT_PT)],
                    agg_hbm.at[pl.ds(cid * FLAT_PAD + sid * FLAT_PT, FLAT_PT)])


def _tc_matmul_body(x_ref, w_ref, o_ref):
    o_ref[...] = jnp.dot(x_ref[...], w_ref[...],
                         preferred_element_type=jnp.float32)


def _tc_norms_body(p_ref, do_ref, di_ref, h_ref, nd_ref, ns_ref):
    do = do_ref[0] + do_ref[1]
    di = di_ref[0] + di_ref[1]
    ns = jnp.where(do > 0.0, lax.rsqrt(do), 0.0)
    nd = jnp.where(di > 0.0, lax.rsqrt(di), 0.0)
    h_ref[...] = p_ref[...] * ns[:, None]
    nd_ref[...] = nd
    ns_ref[...] = ns


def _tc_final_body(agg_ref, nd_ref, ns_ref, b1_ref, w2_ref, b2_ref, o_ref):
    agg = agg_ref[0] + agg_ref[1]
    aggt = agg[:H_FEATS]
    t = agg[H_FEATS]
    r = jnp.maximum(aggt * nd_ref[...][None, :] + b1_ref[...], 0.0)
    c = jnp.sum(r * w2_ref[...], axis=0) * ns_ref[...]
    rowid = lax.broadcasted_iota(jnp.int32, (N_TOT,), 0)
    s = jnp.sum(jnp.where(rowid < N_NODES, c * t, 0.0))
    o_ref[...] = s.reshape(1, 1) / N_NODES + b2_ref[...]


def kernel(in_feat, edge_index, W1, b1, W2, b2):
    xp = jnp.pad(in_feat, ((0, N_TOT - N_NODES), (0, 0)))
    pad_idx = N_NODES + (jnp.arange(E_PAD - N_EDGES, dtype=jnp.int32)
                         % N_PAD_SLOTS)
    src_p = jnp.concatenate([edge_index[0], pad_idx])
    dst_p = jnp.concatenate([edge_index[1], pad_idx])
    src_r = src_p.reshape(NW, NCHUNK, CHUNK)
    dst_r = dst_p.reshape(NW, NCHUNK, CHUNK)
    shift = (jnp.arange(H_FEATS, dtype=jnp.int32) * N_TOT)[None, None, :, None]
    gidx = jnp.concatenate(
        [src_r[:, :, None, :] + shift,
         dst_r[:, :, None, :] + H_FEATS * N_TOT], axis=2)
    sidx = jnp.concatenate(
        [dst_r[:, :, None, :] + shift,
         src_r[:, :, None, :] + H_FEATS * N_TOT], axis=2)
    z1 = jnp.zeros((N_TOT,), jnp.float32)
    zf = jnp.zeros((FLAT_PAD,), jnp.float32)
    ones = jnp.ones((CHUNK,), jnp.float32)

    p = pl.pallas_call(
        _tc_matmul_body,
        out_shape=jax.ShapeDtypeStruct((N_TOT, H_FEATS), jnp.float32),
    )(xp, W1)

    dout_p, din_p = _sc_degrees(src_r, dst_r, z1, ones)

    h1s, nd, ns = pl.pallas_call(
        _tc_norms_body,
        out_shape=[
            jax.ShapeDtypeStruct((N_TOT, H_FEATS), jnp.float32),
            jax.ShapeDtypeStruct((N_TOT,), jnp.float32),
            jax.ShapeDtypeStruct((N_TOT,), jnp.float32),
        ],
    )(p, dout_p.reshape(NC, N_TOT), din_p.reshape(NC, N_TOT))

    tab = jnp.concatenate([h1s.T.reshape(H_FEATS * N_TOT), nd,
                           jnp.zeros((FLAT_PAD - FLAT,), jnp.float32)])

    (agg_p,) = _sc_edge_pass(gidx, sidx, tab, zf)

    out = pl.pallas_call(
        _tc_final_body,
        out_shape=jax.ShapeDtypeStruct((1, 1), jnp.float32),
    )(agg_p.reshape(NC, FLAT_PAD)[:, :FLAT].reshape(NC, NSEG, N_TOT), nd, ns,
      b1.reshape(H_FEATS, 1), W2.reshape(H_FEATS, 1), b2.reshape(1, 1))
    return out


# speedup vs baseline: 17.2258x; 1.1482x over previous
import functools

import jax
import jax.numpy as jnp
from jax import lax
from jax.experimental import pallas as pl
from jax.experimental.pallas import tpu as pltpu
from jax.experimental.pallas import tpu_sc as plsc

N_NODES = 10000
N_EDGES = 320000
IN_FEATS = 128
H_FEATS = 16
NSEG = H_FEATS + 1

NC = 2
NS = 16
NW = NC * NS
CHUNK = 128
NCHUNK = 80
EPT = NCHUNK * CHUNK
E_PAD = NW * EPT
N_PAD_SLOTS = 112
N_TOT = N_NODES + N_PAD_SLOTS
FLAT = NSEG * N_TOT
GA, GB = 8, 9

_MESH = plsc.VectorSubcoreMesh(core_axis_name="c", subcore_axis_name="s")


@functools.partial(
    pl.kernel,
    out_type=[
        jax.ShapeDtypeStruct((NC * N_TOT,), jnp.float32),
        jax.ShapeDtypeStruct((NC * N_TOT,), jnp.float32),
    ],
    mesh=_MESH,
    scratch_types=[
        pltpu.VMEM((NCHUNK, CHUNK), jnp.int32),
        pltpu.VMEM((NCHUNK, CHUNK), jnp.int32),
        pltpu.VMEM((CHUNK,), jnp.float32),
        pltpu.VMEM_SHARED((N_TOT,), jnp.float32),
        pltpu.VMEM_SHARED((N_TOT,), jnp.float32),
    ],
)
def _sc_degrees(src_hbm, dst_hbm, z1_hbm, ones_hbm, dout_hbm, din_hbm,
                src_v, dst_v, ones_v, dout_s, din_s):
    cid = lax.axis_index("c")
    sid = lax.axis_index("s")
    wid = sid * NC + cid

    pltpu.sync_copy(src_hbm.at[wid], src_v)
    pltpu.sync_copy(dst_hbm.at[wid], dst_v)
    pltpu.sync_copy(ones_hbm, ones_v)

    @pl.when(sid == 0)
    def _():
        pltpu.sync_copy(z1_hbm, dout_s)

    @pl.when(sid == 1)
    def _():
        pltpu.sync_copy(z1_hbm, din_s)

    plsc.subcore_barrier()

    def body(j, carry):
        pltpu.sync_copy(ones_v, dout_s.at[src_v.at[j]], add=True)
        pltpu.sync_copy(ones_v, din_s.at[dst_v.at[j]], add=True)
        return carry

    lax.fori_loop(0, NCHUNK, body, 0)
    plsc.subcore_barrier()

    @pl.when(sid == 0)
    def _():
        pltpu.sync_copy(dout_s, dout_hbm.at[pl.ds(cid * N_TOT, N_TOT)])

    @pl.when(sid == 1)
    def _():
        pltpu.sync_copy(din_s, din_hbm.at[pl.ds(cid * N_TOT, N_TOT)])


@functools.partial(
    pl.kernel,
    out_type=[
        jax.ShapeDtypeStruct((NC * FLAT,), jnp.float32),
    ],
    mesh=_MESH,
    scratch_types=(
        [
            pltpu.VMEM((NCHUNK, CHUNK), jnp.int32),
            pltpu.VMEM((NCHUNK, CHUNK), jnp.int32),
            pltpu.VMEM((GB, CHUNK), jnp.float32),
        ]
        + [pltpu.VMEM_SHARED((N_TOT,), jnp.float32)] * NSEG
        + [pltpu.VMEM_SHARED((N_TOT,), jnp.float32)] * NSEG
        + [pltpu.SemaphoreType.DMA, pltpu.SemaphoreType.DMA]
    ),
)
def _sc_edge_pass(src_hbm, dst_hbm, tab_hbm, z1_hbm, agg_hbm, *refs):
    src_v, dst_v, cols_v = refs[0], refs[1], refs[2]
    tab_s = refs[3:3 + NSEG]
    agg_s = refs[3 + NSEG:3 + 2 * NSEG]
    sem_g, sem_s = refs[3 + 2 * NSEG], refs[4 + 2 * NSEG]
    cid = lax.axis_index("c")
    sid = lax.axis_index("s")
    wid = sid * NC + cid

    pltpu.sync_copy(src_hbm.at[wid], src_v)
    pltpu.sync_copy(dst_hbm.at[wid], dst_v)
    for k in range(NSEG):
        @pl.when(sid == k % NS)
        def _(k=k):
            pltpu.sync_copy(tab_hbm.at[pl.ds(k * N_TOT, N_TOT)], tab_s[k])
            pltpu.sync_copy(z1_hbm, agg_s[k])

    plsc.subcore_barrier()

    def run_group(off, size):
        def body(j, carry):
            gath = []
            for k in range(size):
                seg = off + k
                gidx = dst_v if seg == NSEG - 1 else src_v
                gath.append(pltpu.async_copy(tab_s[seg].at[gidx.at[j]],
                                             cols_v.at[k], sem_g))
            for g in gath:
                g.wait()
            scat = []
            for k in range(size):
                seg = off + k
                sidx = src_v if seg == NSEG - 1 else dst_v
                scat.append(pltpu.async_copy(cols_v.at[k],
                                             agg_s[seg].at[sidx.at[j]],
                                             sem_s, add=True))
            for s in scat:
                s.wait()
            return carry

        lax.fori_loop(0, NCHUNK, body, 0)

    run_group(0, GA)
    run_group(GA, GB)
    plsc.subcore_barrier()

    for k in range(NSEG):
        @pl.when(sid == k % NS)
        def _(k=k):
            pltpu.sync_copy(agg_s[k],
                            agg_hbm.at[pl.ds(cid * FLAT + k * N_TOT, N_TOT)])


def _tc_matmul_body(x_ref, w_ref, o_ref):
    o_ref[...] = jnp.dot(x_ref[...], w_ref[...],
                         preferred_element_type=jnp.float32)


def _tc_norms_body(p_ref, do_ref, di_ref, h_ref, nd_ref, ns_ref):
    do = do_ref[0] + do_ref[1]
    di = di_ref[0] + di_ref[1]
    ns = jnp.where(do > 0.0, lax.rsqrt(do), 0.0)
    nd = jnp.where(di > 0.0, lax.rsqrt(di), 0.0)
    h_ref[...] = p_ref[...] * ns[:, None]
    nd_ref[...] = nd
    ns_ref[...] = ns


def _tc_final_body(agg_ref, nd_ref, ns_ref, b1_ref, w2_ref, b2_ref, o_ref):
    agg = agg_ref[0] + agg_ref[1]
    aggt = agg[:H_FEATS]
    t = agg[H_FEATS]
    r = jnp.maximum(aggt * nd_ref[...][None, :] + b1_ref[...], 0.0)
    c = jnp.sum(r * w2_ref[...], axis=0) * ns_ref[...]
    rowid = lax.broadcasted_iota(jnp.int32, (N_TOT,), 0)
    s = jnp.sum(jnp.where(rowid < N_NODES, c * t, 0.0))
    o_ref[...] = s.reshape(1, 1) / N_NODES + b2_ref[...]


def kernel(in_feat, edge_index, W1, b1, W2, b2):
    xp = jnp.pad(in_feat, ((0, N_TOT - N_NODES), (0, 0)))
    pad_idx = N_NODES + (jnp.arange(E_PAD - N_EDGES, dtype=jnp.int32)
                         % N_PAD_SLOTS)
    src_r = jnp.concatenate([edge_index[0], pad_idx]).reshape(NW, NCHUNK, CHUNK)
    dst_r = jnp.concatenate([edge_index[1], pad_idx]).reshape(NW, NCHUNK, CHUNK)
    z1 = jnp.zeros((N_TOT,), jnp.float32)
    ones = jnp.ones((CHUNK,), jnp.float32)

    p = pl.pallas_call(
        _tc_matmul_body,
        out_shape=jax.ShapeDtypeStruct((N_TOT, H_FEATS), jnp.float32),
    )(xp, W1)

    dout_p, din_p = _sc_degrees(src_r, dst_r, z1, ones)

    h1s, nd, ns = pl.pallas_call(
        _tc_norms_body,
        out_shape=[
            jax.ShapeDtypeStruct((N_TOT, H_FEATS), jnp.float32),
            jax.ShapeDtypeStruct((N_TOT,), jnp.float32),
            jax.ShapeDtypeStruct((N_TOT,), jnp.float32),
        ],
    )(p, dout_p.reshape(NC, N_TOT), din_p.reshape(NC, N_TOT))

    tab = jnp.concatenate([h1s.T.reshape(H_FEATS * N_TOT), nd])

    (agg_p,) = _sc_edge_pass(src_r, dst_r, tab, z1)

    out = pl.pallas_call(
        _tc_final_body,
        out_shape=jax.ShapeDtypeStruct((1, 1), jnp.float32),
    )(agg_p.reshape(NC, NSEG, N_TOT), nd, ns,
      b1.reshape(H_FEATS, 1), W2.reshape(H_FEATS, 1), b2.reshape(1, 1))
    return out


# speedup vs baseline: 21.8743x; 1.2699x over previous
import functools

import jax
import jax.numpy as jnp
from jax import lax
from jax.experimental import pallas as pl
from jax.experimental.pallas import tpu as pltpu
from jax.experimental.pallas import tpu_sc as plsc

N_NODES = 10000
N_EDGES = 320000
IN_FEATS = 128
H_FEATS = 16
NSEG = H_FEATS + 1

NC = 2
NS = 16
NW = NC * NS
CHUNK = 128
NCHUNK = 80
EPT = NCHUNK * CHUNK
E_PAD = NW * EPT
N_PAD_SLOTS = 112
N_TOT = N_NODES + N_PAD_SLOTS
FLAT = NSEG * N_TOT
GA, GB = 8, 9

_MESH = plsc.VectorSubcoreMesh(core_axis_name="c", subcore_axis_name="s")


@functools.partial(
    pl.kernel,
    out_type=[
        jax.ShapeDtypeStruct((NC * N_TOT,), jnp.float32),
        jax.ShapeDtypeStruct((NC * N_TOT,), jnp.float32),
    ],
    mesh=_MESH,
    scratch_types=[
        pltpu.VMEM((NCHUNK, CHUNK), jnp.int32),
        pltpu.VMEM((NCHUNK, CHUNK), jnp.int32),
        pltpu.VMEM((CHUNK,), jnp.float32),
        pltpu.VMEM_SHARED((N_TOT,), jnp.float32),
        pltpu.VMEM_SHARED((N_TOT,), jnp.float32),
    ],
)
def _sc_degrees(src_hbm, dst_hbm, z1_hbm, ones_hbm, dout_hbm, din_hbm,
                src_v, dst_v, ones_v, dout_s, din_s):
    cid = lax.axis_index("c")
    sid = lax.axis_index("s")
    wid = sid * NC + cid

    pltpu.sync_copy(src_hbm.at[wid], src_v)
    pltpu.sync_copy(dst_hbm.at[wid], dst_v)
    pltpu.sync_copy(ones_hbm, ones_v)

    @pl.when(sid == 0)
    def _():
        pltpu.sync_copy(z1_hbm, dout_s)

    @pl.when(sid == 1)
    def _():
        pltpu.sync_copy(z1_hbm, din_s)

    plsc.subcore_barrier()

    def body(j, carry):
        pltpu.sync_copy(ones_v, dout_s.at[src_v.at[j]], add=True)
        pltpu.sync_copy(ones_v, din_s.at[dst_v.at[j]], add=True)
        return carry

    lax.fori_loop(0, NCHUNK, body, 0)
    plsc.subcore_barrier()

    @pl.when(sid == 0)
    def _():
        pltpu.sync_copy(dout_s, dout_hbm.at[pl.ds(cid * N_TOT, N_TOT)])

    @pl.when(sid == 1)
    def _():
        pltpu.sync_copy(din_s, din_hbm.at[pl.ds(cid * N_TOT, N_TOT)])


@functools.partial(
    pl.kernel,
    out_type=[
        jax.ShapeDtypeStruct((NC * FLAT,), jnp.float32),
    ],
    mesh=_MESH,
    scratch_types=(
        [
            pltpu.VMEM((NCHUNK, CHUNK), jnp.int32),
            pltpu.VMEM((NCHUNK, CHUNK), jnp.int32),
            pltpu.VMEM((GB, CHUNK), jnp.float32),
        ]
        + [pltpu.VMEM_SHARED((N_TOT,), jnp.float32)] * NSEG
        + [pltpu.VMEM_SHARED((N_TOT,), jnp.float32)] * NSEG
        + [pltpu.SemaphoreType.DMA, pltpu.SemaphoreType.DMA]
    ),
)
def _sc_edge_pass(src_hbm, dst_hbm, tab_hbm, z1_hbm, agg_hbm, *refs):
    src_v, dst_v, cols_v = refs[0], refs[1], refs[2]
    tab_s = refs[3:3 + NSEG]
    agg_s = refs[3 + NSEG:3 + 2 * NSEG]
    sem_g, sem_s = refs[3 + 2 * NSEG], refs[4 + 2 * NSEG]
    cid = lax.axis_index("c")
    sid = lax.axis_index("s")
    wid = sid * NC + cid

    pltpu.sync_copy(src_hbm.at[wid], src_v)
    pltpu.sync_copy(dst_hbm.at[wid], dst_v)
    for k in range(NSEG):
        @pl.when(sid == k % NS)
        def _(k=k):
            pltpu.sync_copy(tab_hbm.at[pl.ds(k * N_TOT, N_TOT)], tab_s[k])
            pltpu.sync_copy(z1_hbm, agg_s[k])

    plsc.subcore_barrier()

    def run_group(off, size):
        def drain_scat(j):
            for k in range(size):
                seg = off + k
                sidx = src_v if seg == NSEG - 1 else dst_v
                pltpu.make_async_copy(cols_v.at[k],
                                      agg_s[seg].at[sidx.at[j]],
                                      sem_s).wait()

        def body(j, carry):
            gath = []
            for k in range(size):
                seg = off + k
                gidx = dst_v if seg == NSEG - 1 else src_v
                sidx = src_v if seg == NSEG - 1 else dst_v

                @pl.when(j > 0)
                def _(k=k, seg=seg, sidx=sidx):
                    pltpu.make_async_copy(cols_v.at[k],
                                          agg_s[seg].at[sidx.at[j]],
                                          sem_s).wait()

                gath.append(pltpu.async_copy(tab_s[seg].at[gidx.at[j]],
                                             cols_v.at[k], sem_g))
            for k in range(size):
                seg = off + k
                sidx = src_v if seg == NSEG - 1 else dst_v
                gath[k].wait()
                pltpu.async_copy(cols_v.at[k], agg_s[seg].at[sidx.at[j]],
                                 sem_s, add=True)
            return carry

        lax.fori_loop(0, NCHUNK, body, 0)
        drain_scat(NCHUNK - 1)

    run_group(0, GA)
    run_group(GA, GB)
    plsc.subcore_barrier()

    for k in range(NSEG):
        @pl.when(sid == k % NS)
        def _(k=k):
            pltpu.sync_copy(agg_s[k],
                            agg_hbm.at[pl.ds(cid * FLAT + k * N_TOT, N_TOT)])


def _tc_matmul_body(x_ref, w_ref, o_ref):
    o_ref[...] = jnp.dot(x_ref[...], w_ref[...],
                         preferred_element_type=jnp.float32)


def _tc_norms_body(p_ref, do_ref, di_ref, h_ref, nd_ref, ns_ref):
    do = do_ref[0] + do_ref[1]
    di = di_ref[0] + di_ref[1]
    ns = jnp.where(do > 0.0, lax.rsqrt(do), 0.0)
    nd = jnp.where(di > 0.0, lax.rsqrt(di), 0.0)
    h_ref[...] = p_ref[...] * ns[:, None]
    nd_ref[...] = nd
    ns_ref[...] = ns


def _tc_final_body(agg_ref, nd_ref, ns_ref, b1_ref, w2_ref, b2_ref, o_ref):
    agg = agg_ref[0] + agg_ref[1]
    aggt = agg[:H_FEATS]
    t = agg[H_FEATS]
    r = jnp.maximum(aggt * nd_ref[...][None, :] + b1_ref[...], 0.0)
    c = jnp.sum(r * w2_ref[...], axis=0) * ns_ref[...]
    rowid = lax.broadcasted_iota(jnp.int32, (N_TOT,), 0)
    s = jnp.sum(jnp.where(rowid < N_NODES, c * t, 0.0))
    o_ref[...] = s.reshape(1, 1) / N_NODES + b2_ref[...]


def kernel(in_feat, edge_index, W1, b1, W2, b2):
    xp = jnp.pad(in_feat, ((0, N_TOT - N_NODES), (0, 0)))
    pad_idx = N_NODES + (jnp.arange(E_PAD - N_EDGES, dtype=jnp.int32)
                         % N_PAD_SLOTS)
    src_r = jnp.concatenate([edge_index[0], pad_idx]).reshape(NW, NCHUNK, CHUNK)
    dst_r = jnp.concatenate([edge_index[1], pad_idx]).reshape(NW, NCHUNK, CHUNK)
    z1 = jnp.zeros((N_TOT,), jnp.float32)
    ones = jnp.ones((CHUNK,), jnp.float32)

    p = pl.pallas_call(
        _tc_matmul_body,
        out_shape=jax.ShapeDtypeStruct((N_TOT, H_FEATS), jnp.float32),
    )(xp, W1)

    dout_p, din_p = _sc_degrees(src_r, dst_r, z1, ones)

    h1s, nd, ns = pl.pallas_call(
        _tc_norms_body,
        out_shape=[
            jax.ShapeDtypeStruct((N_TOT, H_FEATS), jnp.float32),
            jax.ShapeDtypeStruct((N_TOT,), jnp.float32),
            jax.ShapeDtypeStruct((N_TOT,), jnp.float32),
        ],
    )(p, dout_p.reshape(NC, N_TOT), din_p.reshape(NC, N_TOT))

    tab = jnp.concatenate([h1s.T.reshape(H_FEATS * N_TOT), nd])

    (agg_p,) = _sc_edge_pass(src_r, dst_r, tab, z1)

    out = pl.pallas_call(
        _tc_final_body,
        out_shape=jax.ShapeDtypeStruct((1, 1), jnp.float32),
    )(agg_p.reshape(NC, NSEG, N_TOT), nd, ns,
      b1.reshape(H_FEATS, 1), W2.reshape(H_FEATS, 1), b2.reshape(1, 1))
    return out


# speedup vs baseline: 22.3324x; 1.0209x over previous
import functools

import jax
import jax.numpy as jnp
from jax import lax
from jax.experimental import pallas as pl
from jax.experimental.pallas import tpu as pltpu
from jax.experimental.pallas import tpu_sc as plsc

N_NODES = 10000
N_EDGES = 320000
IN_FEATS = 128
H_FEATS = 16
NSEG = H_FEATS + 1

NC = 2
NS = 16
NW = NC * NS
CHUNK = 128
NCHUNK = 80
EPT = NCHUNK * CHUNK
E_PAD = NW * EPT
N_PAD_SLOTS = 112
N_TOT = N_NODES + N_PAD_SLOTS
FLAT = NSEG * N_TOT
GA, GB = 8, 9

_MESH = plsc.VectorSubcoreMesh(core_axis_name="c", subcore_axis_name="s")


@functools.partial(
    pl.kernel,
    out_type=[
        jax.ShapeDtypeStruct((NC * N_TOT,), jnp.float32),
        jax.ShapeDtypeStruct((NC * N_TOT,), jnp.float32),
    ],
    mesh=_MESH,
    scratch_types=[
        pltpu.VMEM((NCHUNK, CHUNK), jnp.int32),
        pltpu.VMEM((NCHUNK, CHUNK), jnp.int32),
        pltpu.VMEM((CHUNK,), jnp.float32),
        pltpu.VMEM_SHARED((N_TOT,), jnp.float32),
        pltpu.VMEM_SHARED((N_TOT,), jnp.float32),
        pltpu.SemaphoreType.DMA,
    ],
)
def _sc_degrees(src_hbm, dst_hbm, z1_hbm, ones_hbm, dout_hbm, din_hbm,
                src_v, dst_v, ones_v, dout_s, din_s, sem_d):
    cid = lax.axis_index("c")
    sid = lax.axis_index("s")
    wid = sid * NC + cid

    pltpu.sync_copy(src_hbm.at[wid], src_v)
    pltpu.sync_copy(dst_hbm.at[wid], dst_v)
    pltpu.sync_copy(ones_hbm, ones_v)

    @pl.when(sid == 0)
    def _():
        pltpu.sync_copy(z1_hbm, dout_s)

    @pl.when(sid == 1)
    def _():
        pltpu.sync_copy(z1_hbm, din_s)

    plsc.subcore_barrier()

    def drain_deg(j):
        pltpu.make_async_copy(ones_v, dout_s.at[src_v.at[j]], sem_d).wait()
        pltpu.make_async_copy(ones_v, din_s.at[dst_v.at[j]], sem_d).wait()

    def body(j, carry):
        @pl.when(j > 0)
        def _():
            drain_deg(j)

        pltpu.async_copy(ones_v, dout_s.at[src_v.at[j]], sem_d, add=True)
        pltpu.async_copy(ones_v, din_s.at[dst_v.at[j]], sem_d, add=True)
        return carry

    lax.fori_loop(0, NCHUNK, body, 0)
    drain_deg(NCHUNK - 1)
    plsc.subcore_barrier()

    @pl.when(sid == 0)
    def _():
        pltpu.sync_copy(dout_s, dout_hbm.at[pl.ds(cid * N_TOT, N_TOT)])

    @pl.when(sid == 1)
    def _():
        pltpu.sync_copy(din_s, din_hbm.at[pl.ds(cid * N_TOT, N_TOT)])


@functools.partial(
    pl.kernel,
    out_type=[
        jax.ShapeDtypeStruct((NC * FLAT,), jnp.float32),
    ],
    mesh=_MESH,
    scratch_types=(
        [
            pltpu.VMEM((NCHUNK, CHUNK), jnp.int32),
            pltpu.VMEM((NCHUNK, CHUNK), jnp.int32),
            pltpu.VMEM((GB, CHUNK), jnp.float32),
        ]
        + [pltpu.VMEM_SHARED((N_TOT,), jnp.float32)] * NSEG
        + [pltpu.VMEM_SHARED((N_TOT,), jnp.float32)] * NSEG
        + [pltpu.SemaphoreType.DMA, pltpu.SemaphoreType.DMA]
    ),
)
def _sc_edge_pass(src_hbm, dst_hbm, tab_hbm, z1_hbm, agg_hbm, *refs):
    src_v, dst_v, cols_v = refs[0], refs[1], refs[2]
    tab_s = refs[3:3 + NSEG]
    agg_s = refs[3 + NSEG:3 + 2 * NSEG]
    sem_g, sem_s = refs[3 + 2 * NSEG], refs[4 + 2 * NSEG]
    cid = lax.axis_index("c")
    sid = lax.axis_index("s")
    wid = sid * NC + cid

    pltpu.sync_copy(src_hbm.at[wid], src_v)
    pltpu.sync_copy(dst_hbm.at[wid], dst_v)
    for k in range(NSEG):
        @pl.when(sid == k % NS)
        def _(k=k):
            pltpu.sync_copy(tab_hbm.at[pl.ds(k * N_TOT, N_TOT)], tab_s[k])
            pltpu.sync_copy(z1_hbm, agg_s[k])

    plsc.subcore_barrier()

    def run_group(off, size):
        def drain_scat(j):
            for k in range(size):
                seg = off + k
                sidx = src_v if seg == NSEG - 1 else dst_v
                pltpu.make_async_copy(cols_v.at[k],
                                      agg_s[seg].at[sidx.at[j]],
                                      sem_s).wait()

        def body(j, carry):
            gath = []
            for k in range(size):
                seg = off + k
                gidx = dst_v if seg == NSEG - 1 else src_v
                sidx = src_v if seg == NSEG - 1 else dst_v

                @pl.when(j > 0)
                def _(k=k, seg=seg, sidx=sidx):
                    pltpu.make_async_copy(cols_v.at[k],
                                          agg_s[seg].at[sidx.at[j]],
                                          sem_s).wait()

                gath.append(pltpu.async_copy(tab_s[seg].at[gidx.at[j]],
                                             cols_v.at[k], sem_g))
            for k in range(size):
                seg = off + k
                sidx = src_v if seg == NSEG - 1 else dst_v
                gath[k].wait()
                pltpu.async_copy(cols_v.at[k], agg_s[seg].at[sidx.at[j]],
                                 sem_s, add=True)
            return carry

        lax.fori_loop(0, NCHUNK, body, 0)
        drain_scat(NCHUNK - 1)

    run_group(0, GA)
    run_group(GA, GB)
    plsc.subcore_barrier()

    for k in range(NSEG):
        @pl.when(sid == k % NS)
        def _(k=k):
            pltpu.sync_copy(agg_s[k],
                            agg_hbm.at[pl.ds(cid * FLAT + k * N_TOT, N_TOT)])


def _tc_matmul_body(x_ref, w_ref, o_ref):
    o_ref[...] = jnp.dot(x_ref[...], w_ref[...],
                         preferred_element_type=jnp.float32)


def _tc_norms_body(p_ref, do_ref, di_ref, h_ref, nd_ref, ns_ref):
    do = do_ref[0] + do_ref[1]
    di = di_ref[0] + di_ref[1]
    ns = jnp.where(do > 0.0, lax.rsqrt(do), 0.0)
    nd = jnp.where(di > 0.0, lax.rsqrt(di), 0.0)
    h_ref[...] = p_ref[...] * ns[:, None]
    nd_ref[...] = nd
    ns_ref[...] = ns


def _tc_final_body(agg_ref, nd_ref, ns_ref, b1_ref, w2_ref, b2_ref, o_ref):
    agg = agg_ref[0] + agg_ref[1]
    aggt = agg[:H_FEATS]
    t = agg[H_FEATS]
    r = jnp.maximum(aggt * nd_ref[...][None, :] + b1_ref[...], 0.0)
    c = jnp.sum(r * w2_ref[...], axis=0) * ns_ref[...]
    rowid = lax.broadcasted_iota(jnp.int32, (N_TOT,), 0)
    s = jnp.sum(jnp.where(rowid < N_NODES, c * t, 0.0))
    o_ref[...] = s.reshape(1, 1) / N_NODES + b2_ref[...]


def kernel(in_feat, edge_index, W1, b1, W2, b2):
    xp = jnp.pad(in_feat, ((0, N_TOT - N_NODES), (0, 0)))
    pad_idx = N_NODES + (jnp.arange(E_PAD - N_EDGES, dtype=jnp.int32)
                         % N_PAD_SLOTS)
    src_r = jnp.concatenate([edge_index[0], pad_idx]).reshape(NW, NCHUNK, CHUNK)
    dst_r = jnp.concatenate([edge_index[1], pad_idx]).reshape(NW, NCHUNK, CHUNK)
    z1 = jnp.zeros((N_TOT,), jnp.float32)
    ones = jnp.ones((CHUNK,), jnp.float32)

    p = pl.pallas_call(
        _tc_matmul_body,
        out_shape=jax.ShapeDtypeStruct((N_TOT, H_FEATS), jnp.float32),
    )(xp, W1)

    dout_p, din_p = _sc_degrees(src_r, dst_r, z1, ones)

    h1s, nd, ns = pl.pallas_call(
        _tc_norms_body,
        out_shape=[
            jax.ShapeDtypeStruct((N_TOT, H_FEATS), jnp.float32),
            jax.ShapeDtypeStruct((N_TOT,), jnp.float32),
            jax.ShapeDtypeStruct((N_TOT,), jnp.float32),
        ],
    )(p, dout_p.reshape(NC, N_TOT), din_p.reshape(NC, N_TOT))

    tab = jnp.concatenate([h1s.T.reshape(H_FEATS * N_TOT), nd])

    (agg_p,) = _sc_edge_pass(src_r, dst_r, tab, z1)

    out = pl.pallas_call(
        _tc_final_body,
        out_shape=jax.ShapeDtypeStruct((1, 1), jnp.float32),
    )(agg_p.reshape(NC, NSEG, N_TOT), nd, ns,
      b1.reshape(H_FEATS, 1), W2.reshape(H_FEATS, 1), b2.reshape(1, 1))
    return out
